# baseline (speedup 1.0000x reference)
import functools

import jax
import jax.numpy as jnp
from jax import lax
from jax.experimental import pallas as pl
from jax.experimental.pallas import tpu as pltpu
from jax.experimental.pallas import tpu_sc as plsc

_T = 4096
_D = 1024
_NC = 2
_NS = 16
_NW = _NC * _NS
_RPW = _T // _NW
_HALF = _RPW // 2
_CH = 16
_NBUF = 4
_NCH = _HALF // _CH


def _make_sc_copy():
    mesh = plsc.VectorSubcoreMesh(core_axis_name="c", subcore_axis_name="s")

    @functools.partial(
        pl.kernel,
        mesh=mesh,
        out_type=jax.ShapeDtypeStruct((_T, _D), jnp.float32),
        scratch_types=[
            pltpu.VMEM((_NBUF, _CH, _D), jnp.float32),
            pltpu.VMEM_SHARED((_NS, _NBUF, _CH, _D), jnp.float32),
            *([pltpu.SemaphoreType.DMA] * (4 * _NBUF)),
        ],
    )
    def sc_copy(table_hbm, out_hbm, tbuf, sbuf, *sems):
        t_in = sems[0:_NBUF]
        t_out = sems[_NBUF : 2 * _NBUF]
        s_in = sems[2 * _NBUF : 3 * _NBUF]
        s_out = sems[3 * _NBUF : 4 * _NBUF]
        wid = lax.axis_index("s") * _NC + lax.axis_index("c")
        sid = lax.axis_index("s")
        base_t = wid * _RPW
        base_s = base_t + _HALF

        def t_fire_in(k):
            b = k % _NBUF
            return pltpu.async_copy(
                table_hbm.at[pl.ds(base_t + k * _CH, _CH)], tbuf.at[b], t_in[b]
            )

        def t_fire_out(k):
            b = k % _NBUF
            return pltpu.async_copy(
                tbuf.at[b], out_hbm.at[pl.ds(base_t + k * _CH, _CH)], t_out[b]
            )

        def s_fire_in(k):
            b = k % _NBUF
            return pltpu.async_copy(
                table_hbm.at[pl.ds(base_s + k * _CH, _CH)], sbuf.at[sid, b], s_in[b]
            )

        def s_fire_out(k):
            b = k % _NBUF
            return pltpu.async_copy(
                sbuf.at[sid, b], out_hbm.at[pl.ds(base_s + k * _CH, _CH)], s_out[b]
            )

        t_icp = [None] * _NCH
        t_ocp = [None] * _NCH
        s_icp = [None] * _NCH
        s_ocp = [None] * _NCH
        for j in range(min(_NBUF, _NCH)):
            t_icp[j] = t_fire_in(j)
            s_icp[j] = s_fire_in(j)
        for k in range(_NCH):
            if k >= _NBUF:
                t_ocp[k - _NBUF].wait()
                t_icp[k] = t_fire_in(k)
                s_ocp[k - _NBUF].wait()
                s_icp[k] = s_fire_in(k)
            t_icp[k].wait()
            t_ocp[k] = t_fire_out(k)
            s_icp[k].wait()
            s_ocp[k] = s_fire_out(k)
        for k in range(max(0, _NCH - _NBUF), _NCH):
            t_ocp[k].wait()
            s_ocp[k].wait()

    return sc_copy


_sc_copy = _make_sc_copy()


def kernel(x, pe_table):
    del x
    out = _sc_copy(pe_table)
    return out[None]


# speedup vs baseline: 1.0174x; 1.0174x over previous
import functools

import jax
import jax.numpy as jnp
from jax import lax
from jax.experimental import pallas as pl
from jax.experimental.pallas import tpu as pltpu
from jax.experimental.pallas import tpu_sc as plsc

_T = 4096
_D = 1024
_NC = 2
_NS = 16
_NW = _NC * _NS
_RPW = _T // _NW
_HALF = _RPW // 2
_CH = 32
_NBUF = 2
_NCH = _HALF // _CH


def _make_sc_copy():
    mesh = plsc.VectorSubcoreMesh(core_axis_name="c", subcore_axis_name="s")

    @functools.partial(
        pl.kernel,
        mesh=mesh,
        out_type=jax.ShapeDtypeStruct((_T, _D), jnp.float32),
        scratch_types=[
            pltpu.VMEM((_NBUF, _CH, _D), jnp.float32),
            pltpu.VMEM_SHARED((_NS, _NBUF, _CH, _D), jnp.float32),
            *([pltpu.SemaphoreType.DMA] * (4 * _NBUF)),
        ],
    )
    def sc_copy(table_hbm, out_hbm, tbuf, sbuf, *sems):
        t_in = sems[0:_NBUF]
        t_out = sems[_NBUF : 2 * _NBUF]
        s_in = sems[2 * _NBUF : 3 * _NBUF]
        s_out = sems[3 * _NBUF : 4 * _NBUF]
        wid = lax.axis_index("s") * _NC + lax.axis_index("c")
        sid = lax.axis_index("s")
        base_t = wid * _RPW
        base_s = base_t + _HALF

        def t_fire_in(k):
            b = k % _NBUF
            return pltpu.async_copy(
                table_hbm.at[pl.ds(base_t + k * _CH, _CH)], tbuf.at[b], t_in[b]
            )

        def t_fire_out(k):
            b = k % _NBUF
            return pltpu.async_copy(
                tbuf.at[b], out_hbm.at[pl.ds(base_t + k * _CH, _CH)], t_out[b]
            )

        def s_fire_in(k):
            b = k % _NBUF
            return pltpu.async_copy(
                table_hbm.at[pl.ds(base_s + k * _CH, _CH)], sbuf.at[sid, b], s_in[b]
            )

        def s_fire_out(k):
            b = k % _NBUF
            return pltpu.async_copy(
                sbuf.at[sid, b], out_hbm.at[pl.ds(base_s + k * _CH, _CH)], s_out[b]
            )

        t_icp = [None] * _NCH
        t_ocp = [None] * _NCH
        s_icp = [None] * _NCH
        s_ocp = [None] * _NCH
        for j in range(min(_NBUF, _NCH)):
            t_icp[j] = t_fire_in(j)
            s_icp[j] = s_fire_in(j)
        for k in range(_NCH):
            if k >= _NBUF:
                t_ocp[k - _NBUF].wait()
                t_icp[k] = t_fire_in(k)
                s_ocp[k - _NBUF].wait()
                s_icp[k] = s_fire_in(k)
            t_icp[k].wait()
            t_ocp[k] = t_fire_out(k)
            s_icp[k].wait()
            s_ocp[k] = s_fire_out(k)
        for k in range(max(0, _NCH - _NBUF), _NCH):
            t_ocp[k].wait()
            s_ocp[k].wait()

    return sc_copy


_sc_copy = _make_sc_copy()


def kernel(x, pe_table):
    del x
    out = _sc_copy(pe_table)
    return out[None]
